# baseline (speedup 1.0000x reference)
import jax
import jax.numpy as jnp
from jax.experimental import pallas as pl
from jax.experimental.pallas import tpu as pltpu

_BATCH = 16
_N_NODES = 512
_HIDDEN = 768
_NBUF = 4


def _body(node, tok_ref, out, ibuf, obuf, sem_in, sem_out):
    def cp_in(b):
        return pltpu.make_async_copy(node.at[b], ibuf.at[b % _NBUF],
                                     sem_in.at[b % _NBUF])

    def cp_out(b):
        return pltpu.make_async_copy(obuf.at[b % _NBUF], out.at[b],
                                     sem_out.at[b % _NBUF])

    for b in range(_NBUF):
        cp_in(b).start()
    for b in range(_BATCH):
        s = b % _NBUF
        cp_in(b).wait()
        if b >= _NBUF:
            cp_out(b - _NBUF).wait()
        obuf[s, 0:1, :] = tok_ref[...]
        obuf[s, 1:_N_NODES + 1, :] = ibuf[s]
        cp_out(b).start()
        if b + _NBUF < _BATCH:
            cp_in(b + _NBUF).start()
    for b in range(_BATCH - _NBUF, _BATCH):
        cp_out(b).wait()


@jax.jit
def _tc_call(node_feature, graph_token):
    return pl.pallas_call(
        _body,
        in_specs=[
            pl.BlockSpec(memory_space=pl.ANY),
            pl.BlockSpec(memory_space=pltpu.VMEM),
        ],
        out_specs=pl.BlockSpec(memory_space=pl.ANY),
        out_shape=jax.ShapeDtypeStruct((_BATCH, _N_NODES + 1, _HIDDEN),
                                       jnp.float32),
        scratch_shapes=[
            pltpu.VMEM((_NBUF, _N_NODES, _HIDDEN), jnp.float32),
            pltpu.VMEM((_NBUF, _N_NODES + 1, _HIDDEN), jnp.float32),
            pltpu.SemaphoreType.DMA((_NBUF,)),
            pltpu.SemaphoreType.DMA((_NBUF,)),
        ],
        compiler_params=pltpu.CompilerParams(
            vmem_limit_bytes=100 * 1024 * 1024,
        ),
    )(node_feature, graph_token)


def kernel(node_feature, graph_token):
    return _tc_call(node_feature, graph_token)


# speedup vs baseline: 1.0420x; 1.0420x over previous
import jax
import jax.numpy as jnp
from jax.experimental import pallas as pl
from jax.experimental.pallas import tpu as pltpu

_BATCH = 16
_N_NODES = 512
_HIDDEN = 768
_BB = 8


def _tc_body(node_ref, tok_ref, out_ref):
    for i in range(_BB):
        out_ref[i, 0:1, :] = tok_ref[...]
        out_ref[i, 1:_N_NODES + 1, :] = node_ref[i]


@jax.jit
def _tc_call(node_feature, graph_token):
    return pl.pallas_call(
        _tc_body,
        grid=(_BATCH // _BB,),
        in_specs=[
            pl.BlockSpec((_BB, _N_NODES, _HIDDEN), lambda b: (b, 0, 0)),
            pl.BlockSpec((1, _HIDDEN), lambda b: (0, 0)),
        ],
        out_specs=pl.BlockSpec((_BB, _N_NODES + 1, _HIDDEN),
                               lambda b: (b, 0, 0)),
        out_shape=jax.ShapeDtypeStruct((_BATCH, _N_NODES + 1, _HIDDEN),
                                       jnp.float32),
        compiler_params=pltpu.CompilerParams(
            dimension_semantics=("parallel",),
        ),
    )(node_feature, graph_token)


def kernel(node_feature, graph_token):
    return _tc_call(node_feature, graph_token)
